# baseline (speedup 1.0000x reference)
import functools

import jax
import jax.numpy as jnp
from jax import lax
from jax.experimental import pallas as pl
from jax.experimental.pallas import tpu as pltpu
from jax.experimental.pallas import tpu_sc as plsc

_INFO = plsc.get_sparse_core_info()
_NC, _NS = _INFO.num_cores, _INFO.num_subcores
_NW = _NC * _NS
_L = _INFO.num_lanes
_KU = 16


@functools.partial(jax.jit, static_argnames=("n_rows", "d_model"))
def _sc_embedding_lookup(weight, idx_flat, *, n_rows, d_model):
    n_types = weight.shape[0]
    b_per_w = n_rows // _NW
    n_groups = b_per_w // _L
    g_elems = _L * d_model
    mesh = plsc.VectorSubcoreMesh(core_axis_name="c", subcore_axis_name="s")

    @functools.partial(
        pl.kernel,
        out_type=jax.ShapeDtypeStruct((n_rows * d_model,), jnp.float32),
        mesh=mesh,
        compiler_params=pltpu.CompilerParams(needs_layout_passes=False),
        scratch_types=[
            pltpu.VMEM((b_per_w,), jnp.int32),
            pltpu.VMEM((n_types * d_model,), jnp.float32),
            pltpu.VMEM((g_elems,), jnp.float32),
            pltpu.VMEM((g_elems,), jnp.float32),
            pltpu.SemaphoreType.DMA,
            pltpu.SemaphoreType.DMA,
        ],
    )
    def run(table_hbm, idx_hbm, out_hbm, idx_v, table_f, sb0, sb1, os0, os1):
        wid = lax.axis_index("s") * _NC + lax.axis_index("c")
        base = wid * b_per_w
        pltpu.sync_copy(table_hbm, table_f)
        pltpu.sync_copy(idx_hbm.at[pl.ds(base, b_per_w)], idx_v)
        diota = lax.iota(jnp.int32, _L) * d_model

        def fill(sbuf, g):
            tvec = idx_v[pl.ds(g * _L, _L)]
            src0 = tvec * d_model

            @pl.loop(0, d_model // _KU)
            def _cols(jj):
                sbase = src0 + jj * _KU
                dbase = diota + jj * _KU
                for k in range(_KU):
                    v = plsc.load_gather(table_f, [sbase + k])
                    plsc.store_scatter(sbuf, [dbase + k], v)

        def put(sbuf, g, sem):
            return pltpu.async_copy(
                sbuf, out_hbm.at[pl.ds((base + g * _L) * d_model, g_elems)],
                sem)

        def put_wait(sbuf, sem):
            pltpu.make_async_copy(
                sbuf, out_hbm.at[pl.ds(base * d_model, g_elems)], sem).wait()

        fill(sb0, 0)
        put(sb0, 0, os0)
        fill(sb1, 1)
        put(sb1, 1, os1)

        @pl.loop(1, n_groups // 2)
        def _pair(p):
            g0 = p * 2
            put_wait(sb0, os0)
            fill(sb0, g0)
            put(sb0, g0, os0)
            put_wait(sb1, os1)
            fill(sb1, g0 + 1)
            put(sb1, g0 + 1, os1)

        put_wait(sb0, os0)
        put_wait(sb1, os1)

    return run(weight.reshape(-1), idx_flat)


def kernel(token_types, weight):
    n_rows = token_types.size
    d_model = weight.shape[1]
    idx_flat = token_types.reshape(-1).astype(jnp.int32)
    out = _sc_embedding_lookup(weight, idx_flat, n_rows=n_rows,
                               d_model=d_model)
    return out.reshape(token_types.shape + (d_model,))


# speedup vs baseline: 2.1061x; 2.1061x over previous
import functools

import jax
import jax.numpy as jnp
from jax import lax
from jax.experimental import pallas as pl
from jax.experimental.pallas import tpu as pltpu
from jax.experimental.pallas import tpu_sc as plsc

_INFO = plsc.get_sparse_core_info()
_NC, _NS = _INFO.num_cores, _INFO.num_subcores
_NW = _NC * _NS
_L = _INFO.num_lanes
_KU = 16


@functools.partial(jax.jit, static_argnames=("n_rows", "d_model"))
def _sc_embedding_lookup(weight, idx_flat, *, n_rows, d_model):
    n_types = weight.shape[0]
    b_per_w = n_rows // _NW
    n_groups = b_per_w // _L
    g_elems = _L * d_model
    mesh = plsc.VectorSubcoreMesh(core_axis_name="c", subcore_axis_name="s")

    @functools.partial(
        pl.kernel,
        out_type=jax.ShapeDtypeStruct((n_rows * d_model,), jnp.float32),
        mesh=mesh,
        compiler_params=pltpu.CompilerParams(needs_layout_passes=False),
        scratch_types=[
            pltpu.VMEM((b_per_w,), jnp.int32),
            pltpu.VMEM((n_types * d_model,), jnp.float32),
            pltpu.VMEM((g_elems,), jnp.float32),
            pltpu.VMEM((g_elems,), jnp.float32),
            pltpu.SemaphoreType.DMA,
            pltpu.SemaphoreType.DMA,
        ],
    )
    def run(table_hbm, idx_hbm, out_hbm, idx_v, table_f, sb0, sb1, os0, os1):
        wid = lax.axis_index("s") * _NC + lax.axis_index("c")
        base = wid * b_per_w
        pltpu.sync_copy(table_hbm, table_f)
        pltpu.sync_copy(idx_hbm.at[pl.ds(base, b_per_w)], idx_v)
        diota = lax.iota(jnp.int32, _L) * d_model

        def fill(sbuf, g):
            tvec = idx_v[pl.ds(g * _L, _L)]
            src0 = tvec * d_model

            @plsc.parallel_loop(0, d_model, step=_KU, unroll=4)
            def _cols(jj):
                sbase = src0 + jj
                dbase = diota + jj
                for k in range(_KU):
                    v = plsc.load_gather(table_f, [sbase + k])
                    plsc.store_scatter(sbuf, [dbase + k], v)

        def put(sbuf, g, sem):
            return pltpu.async_copy(
                sbuf, out_hbm.at[pl.ds((base + g * _L) * d_model, g_elems)],
                sem)

        def put_wait(sbuf, sem):
            pltpu.make_async_copy(
                sbuf, out_hbm.at[pl.ds(base * d_model, g_elems)], sem).wait()

        fill(sb0, 0)
        put(sb0, 0, os0)
        fill(sb1, 1)
        put(sb1, 1, os1)

        @pl.loop(1, n_groups // 2)
        def _pair(p):
            g0 = p * 2
            put_wait(sb0, os0)
            fill(sb0, g0)
            put(sb0, g0, os0)
            put_wait(sb1, os1)
            fill(sb1, g0 + 1)
            put(sb1, g0 + 1, os1)

        put_wait(sb0, os0)
        put_wait(sb1, os1)

    return run(weight.reshape(-1), idx_flat)


def kernel(token_types, weight):
    n_rows = token_types.size
    d_model = weight.shape[1]
    idx_flat = token_types.reshape(-1).astype(jnp.int32)
    out = _sc_embedding_lookup(weight, idx_flat, n_rows=n_rows,
                               d_model=d_model)
    return out.reshape(token_types.shape + (d_model,))


# speedup vs baseline: 6.0004x; 2.8491x over previous
import functools

import jax
import jax.numpy as jnp
from jax import lax
from jax.experimental import pallas as pl
from jax.experimental.pallas import tpu as pltpu
from jax.experimental.pallas import tpu_sc as plsc

_INFO = plsc.get_sparse_core_info()
_NC, _NS = _INFO.num_cores, _INFO.num_subcores
_NW = _NC * _NS
_L = _INFO.num_lanes
_KU = 16


@functools.partial(jax.jit, static_argnames=("n_rows", "d_model"))
def _sc_embedding_lookup(weight, idx_flat, *, n_rows, d_model):
    n_types = weight.shape[0]
    b_per_w = n_rows // _NW
    n_groups = b_per_w // _L
    g_elems = _L * d_model
    mesh = plsc.VectorSubcoreMesh(core_axis_name="c", subcore_axis_name="s")

    @functools.partial(
        pl.kernel,
        out_type=jax.ShapeDtypeStruct((n_rows * d_model,), jnp.float32),
        mesh=mesh,
        compiler_params=pltpu.CompilerParams(needs_layout_passes=False),
        scratch_types=[
            pltpu.VMEM((b_per_w,), jnp.int32),
            pltpu.VMEM((n_types * d_model,), jnp.float32),
            pltpu.VMEM((g_elems,), jnp.float32),
            pltpu.VMEM((g_elems,), jnp.float32),
            pltpu.SemaphoreType.DMA,
            pltpu.SemaphoreType.DMA,
        ],
    )
    def run(table_hbm, idx_hbm, out_hbm, idx_v, table_f, sb0, sb1, os0, os1):
        wid = lax.axis_index("s") * _NC + lax.axis_index("c")
        base = wid * b_per_w
        pltpu.sync_copy(table_hbm, table_f)
        pltpu.sync_copy(idx_hbm.at[pl.ds(base, b_per_w)], idx_v)
        lanes = lax.iota(jnp.int32, _L)

        def fill(sbuf, g):
            tvec = idx_v[pl.ds(g * _L, _L)]

            @pl.loop(0, _L)
            def _tok(t):
                row = jnp.sum(jnp.where(lanes == t, tvec, 0))
                src0 = row * d_model
                dst0 = t * d_model

                @plsc.parallel_loop(0, d_model, step=4 * _L, unroll=2)
                def _cols(jj):
                    for k in range(4):
                        sl = jj + k * _L
                        sbuf[pl.ds(dst0 + sl, _L)] = (
                            table_f[pl.ds(src0 + sl, _L)])

        def put(sbuf, g, sem):
            return pltpu.async_copy(
                sbuf, out_hbm.at[pl.ds((base + g * _L) * d_model, g_elems)],
                sem)

        def put_wait(sbuf, sem):
            pltpu.make_async_copy(
                sbuf, out_hbm.at[pl.ds(base * d_model, g_elems)], sem).wait()

        fill(sb0, 0)
        put(sb0, 0, os0)
        fill(sb1, 1)
        put(sb1, 1, os1)

        @pl.loop(1, n_groups // 2)
        def _pair(p):
            g0 = p * 2
            put_wait(sb0, os0)
            fill(sb0, g0)
            put(sb0, g0, os0)
            put_wait(sb1, os1)
            fill(sb1, g0 + 1)
            put(sb1, g0 + 1, os1)

        put_wait(sb0, os0)
        put_wait(sb1, os1)

    return run(weight.reshape(-1), idx_flat)


def kernel(token_types, weight):
    n_rows = token_types.size
    d_model = weight.shape[1]
    idx_flat = token_types.reshape(-1).astype(jnp.int32)
    out = _sc_embedding_lookup(weight, idx_flat, n_rows=n_rows,
                               d_model=d_model)
    return out.reshape(token_types.shape + (d_model,))


# speedup vs baseline: 6.7179x; 1.1196x over previous
import functools

import jax
import jax.numpy as jnp
from jax import lax
from jax.experimental import pallas as pl
from jax.experimental.pallas import tpu as pltpu
from jax.experimental.pallas import tpu_sc as plsc

_INFO = plsc.get_sparse_core_info()
_NC, _NS = _INFO.num_cores, _INFO.num_subcores
_NW = _NC * _NS
_L = _INFO.num_lanes
_KU = 16


@functools.partial(jax.jit, static_argnames=("n_rows", "d_model"))
def _sc_embedding_lookup(weight, idx_flat, *, n_rows, d_model):
    n_types = weight.shape[0]
    b_per_w = n_rows // _NW
    n_groups = b_per_w // _L
    g_elems = _L * d_model
    mesh = plsc.VectorSubcoreMesh(core_axis_name="c", subcore_axis_name="s")

    @functools.partial(
        pl.kernel,
        out_type=jax.ShapeDtypeStruct((n_rows * d_model,), jnp.float32),
        mesh=mesh,
        compiler_params=pltpu.CompilerParams(needs_layout_passes=False),
        scratch_types=[
            pltpu.VMEM((b_per_w,), jnp.int32),
            pltpu.VMEM((n_types * d_model,), jnp.float32),
            pltpu.SemaphoreType.DMA,
        ],
    )
    def run(table_hbm, idx_hbm, out_hbm, idx_v, table_f, osem):
        wid = lax.axis_index("s") * _NC + lax.axis_index("c")
        base = wid * b_per_w
        pltpu.sync_copy(table_hbm, table_f)
        pltpu.sync_copy(idx_hbm.at[pl.ds(base, b_per_w)], idx_v)
        lanes = lax.iota(jnp.int32, _L)

        @pl.loop(0, n_groups)
        def _grp(g):
            tvec = idx_v[pl.ds(g * _L, _L)]
            for t in range(_L):
                row = jnp.sum(jnp.where(lanes == t, tvec, 0))
                i = g * _L + t
                pltpu.async_copy(
                    table_f.at[pl.ds(row * d_model, d_model)],
                    out_hbm.at[pl.ds((base + i) * d_model, d_model)],
                    osem)

        @pl.loop(0, b_per_w)
        def _drain(i):
            pltpu.make_async_copy(
                table_f.at[pl.ds(0, d_model)],
                out_hbm.at[pl.ds(base * d_model, d_model)],
                osem).wait()

    return run(weight.reshape(-1), idx_flat)


def kernel(token_types, weight):
    n_rows = token_types.size
    d_model = weight.shape[1]
    idx_flat = token_types.reshape(-1).astype(jnp.int32)
    out = _sc_embedding_lookup(weight, idx_flat, n_rows=n_rows,
                               d_model=d_model)
    return out.reshape(token_types.shape + (d_model,))


# speedup vs baseline: 6.8365x; 1.0177x over previous
import functools

import jax
import jax.numpy as jnp
from jax import lax
from jax.experimental import pallas as pl
from jax.experimental.pallas import tpu as pltpu
from jax.experimental.pallas import tpu_sc as plsc

_INFO = plsc.get_sparse_core_info()
_NC, _NS = _INFO.num_cores, _INFO.num_subcores
_NW = _NC * _NS
_L = _INFO.num_lanes
_KU = 16


@functools.partial(jax.jit, static_argnames=("n_rows", "d_model"))
def _sc_embedding_lookup(weight, idx_flat, *, n_rows, d_model):
    n_types = weight.shape[0]
    b_per_w = n_rows // _NW
    n_groups = b_per_w // _L
    g_elems = _L * d_model
    mesh = plsc.VectorSubcoreMesh(core_axis_name="c", subcore_axis_name="s")

    @functools.partial(
        pl.kernel,
        out_type=jax.ShapeDtypeStruct((n_rows * d_model,), jnp.float32),
        mesh=mesh,
        compiler_params=pltpu.CompilerParams(needs_layout_passes=False),
        scratch_types=[
            pltpu.VMEM((b_per_w,), jnp.int32),
            pltpu.VMEM((n_types * d_model,), jnp.float32),
            *[pltpu.SemaphoreType.DMA for _ in range(4)],
        ],
    )
    def run(table_hbm, idx_hbm, out_hbm, idx_v, table_f, *osems):
        wid = lax.axis_index("s") * _NC + lax.axis_index("c")
        base = wid * b_per_w
        pltpu.sync_copy(table_hbm, table_f)
        pltpu.sync_copy(idx_hbm.at[pl.ds(base, b_per_w)], idx_v)
        lanes = lax.iota(jnp.int32, _L)

        @pl.loop(0, n_groups)
        def _grp(g):
            tvec = idx_v[pl.ds(g * _L, _L)]
            for t in range(_L):
                row = jnp.sum(jnp.where(lanes == t, tvec, 0))
                i = g * _L + t
                pltpu.async_copy(
                    table_f.at[pl.ds(row * d_model, d_model)],
                    out_hbm.at[pl.ds((base + i) * d_model, d_model)],
                    osems[t % 4])

        @pl.loop(0, b_per_w // 4)
        def _drain(i):
            for q in range(4):
                pltpu.make_async_copy(
                    table_f.at[pl.ds(0, d_model)],
                    out_hbm.at[pl.ds(base * d_model, d_model)],
                    osems[q]).wait()

    return run(weight.reshape(-1), idx_flat)


def kernel(token_types, weight):
    n_rows = token_types.size
    d_model = weight.shape[1]
    idx_flat = token_types.reshape(-1).astype(jnp.int32)
    out = _sc_embedding_lookup(weight, idx_flat, n_rows=n_rows,
                               d_model=d_model)
    return out.reshape(token_types.shape + (d_model,))
